# Initial kernel scaffold; baseline (speedup 1.0000x reference)
import jax
import jax.numpy as jnp
from jax.experimental import pallas as pl


def kernel(z, w1, b1, w2, b2):
    raise NotImplementedError("write your pallas kernel here")


# speedup vs baseline: 2.1866x; 2.1866x over previous
import jax
import jax.numpy as jnp
from jax import lax
from jax.experimental import pallas as pl
from jax.experimental.pallas import tpu as pltpu


def _leaky(x, slope=0.01):
    return jnp.where(x > 0, x, slope * x)


def _mlp_kernel(z_ref, w1_ref, b1_ref, w2_ref, b2_ref, t_ref):
    z = z_ref[...]
    h = _leaky(jnp.dot(z, w1_ref[...], preferred_element_type=jnp.float32)
               + b1_ref[...])
    t = _leaky(jnp.dot(h, w2_ref[...], preferred_element_type=jnp.float32)
               + b2_ref[...])
    t_ref[...] = t.astype(jnp.bfloat16)


def _gram_kernel(tr_ref, tc_ref, out_ref):
    out_ref[...] = lax.dot_general(
        tr_ref[...], tc_ref[...],
        dimension_numbers=(((1,), (1,)), ((), ())),
        preferred_element_type=jnp.float32)


def kernel(z, w1, b1, w2, b2):
    z = z.astype(jnp.float32)
    N, D = z.shape
    R, _, H1 = w1.shape
    H2 = w2.shape[2]
    HP = 128

    w1p = jnp.pad(jnp.transpose(w1, (1, 0, 2)).reshape(D, R * H1),
                  ((0, 0), (0, HP - R * H1))).astype(jnp.float32)
    b1p = jnp.pad(b1.reshape(1, R * H1), ((0, 0), (0, HP - R * H1))).astype(jnp.float32)
    w2p = jnp.zeros((HP, HP), jnp.float32)
    for r in range(R):
        w2p = w2p.at[r * H1:(r + 1) * H1, r * H2:(r + 1) * H2].set(
            w2[r].astype(jnp.float32))
    b2p = jnp.pad(b2.reshape(1, R * H2), ((0, 0), (0, HP - R * H2))).astype(jnp.float32)

    TM1 = 1024
    t_mat = pl.pallas_call(
        _mlp_kernel,
        out_shape=jax.ShapeDtypeStruct((N, HP), jnp.bfloat16),
        grid=(N // TM1,),
        in_specs=[
            pl.BlockSpec((TM1, D), lambda i: (i, 0)),
            pl.BlockSpec((D, HP), lambda i: (0, 0)),
            pl.BlockSpec((1, HP), lambda i: (0, 0)),
            pl.BlockSpec((HP, HP), lambda i: (0, 0)),
            pl.BlockSpec((1, HP), lambda i: (0, 0)),
        ],
        out_specs=pl.BlockSpec((TM1, HP), lambda i: (i, 0)),
        compiler_params=pltpu.CompilerParams(dimension_semantics=("parallel",)),
    )(z, w1p, b1p, w2p, b2p)

    TM, TN = 1024, 1024
    out = pl.pallas_call(
        _gram_kernel,
        out_shape=jax.ShapeDtypeStruct((N, N), jnp.float32),
        grid=(N // TM, N // TN),
        in_specs=[
            pl.BlockSpec((TM, HP), lambda i, j: (i, 0)),
            pl.BlockSpec((TN, HP), lambda i, j: (j, 0)),
        ],
        out_specs=pl.BlockSpec((TM, TN), lambda i, j: (i, j)),
        compiler_params=pltpu.CompilerParams(
            dimension_semantics=("parallel", "parallel")),
        cost_estimate=pl.CostEstimate(
            flops=2 * N * N * HP, transcendentals=0,
            bytes_accessed=4 * N * N + 2 * 2 * N * HP),
    )(t_mat, t_mat)
    return out


# speedup vs baseline: 2.5550x; 1.1685x over previous
import jax
import jax.numpy as jnp
from jax import lax
from jax.experimental import pallas as pl
from jax.experimental.pallas import tpu as pltpu


def _leaky(x, slope=0.01):
    return jnp.where(x > 0, x, slope * x)


def _mlp_kernel(z_ref, w1_ref, b1_ref, w2_ref, b2_ref, t_ref):
    z = z_ref[...]
    h = _leaky(jnp.dot(z, w1_ref[...], preferred_element_type=jnp.float32)
               + b1_ref[...])
    t = _leaky(jnp.dot(h, w2_ref[...], preferred_element_type=jnp.float32)
               + b2_ref[...])
    t_ref[...] = t.astype(jnp.bfloat16)


def _gram_kernel(tr_ref, tc_ref, out_ref):
    out_ref[...] = lax.dot_general(
        tr_ref[...], tc_ref[...],
        dimension_numbers=(((1,), (1,)), ((), ())),
        preferred_element_type=jnp.float32)


def kernel(z, w1, b1, w2, b2):
    z = z.astype(jnp.float32)
    N, D = z.shape
    R, _, H1 = w1.shape
    H2 = w2.shape[2]
    HP = 128

    w1p = jnp.pad(jnp.transpose(w1, (1, 0, 2)).reshape(D, R * H1),
                  ((0, 0), (0, HP - R * H1))).astype(jnp.float32)
    b1p = jnp.pad(b1.reshape(1, R * H1), ((0, 0), (0, HP - R * H1))).astype(jnp.float32)
    w2p = jnp.zeros((HP, HP), jnp.float32)
    for r in range(R):
        w2p = w2p.at[r * H1:(r + 1) * H1, r * H2:(r + 1) * H2].set(
            w2[r].astype(jnp.float32))
    b2p = jnp.pad(b2.reshape(1, R * H2), ((0, 0), (0, HP - R * H2))).astype(jnp.float32)

    TM1 = 1024
    t_mat = pl.pallas_call(
        _mlp_kernel,
        out_shape=jax.ShapeDtypeStruct((N, HP), jnp.bfloat16),
        grid=(N // TM1,),
        in_specs=[
            pl.BlockSpec((TM1, D), lambda i: (i, 0)),
            pl.BlockSpec((D, HP), lambda i: (0, 0)),
            pl.BlockSpec((1, HP), lambda i: (0, 0)),
            pl.BlockSpec((HP, HP), lambda i: (0, 0)),
            pl.BlockSpec((1, HP), lambda i: (0, 0)),
        ],
        out_specs=pl.BlockSpec((TM1, HP), lambda i: (i, 0)),
        compiler_params=pltpu.CompilerParams(dimension_semantics=("parallel",)),
    )(z, w1p, b1p, w2p, b2p)

    TM = 512
    out = pl.pallas_call(
        _gram_kernel,
        out_shape=jax.ShapeDtypeStruct((N, N), jnp.float32),
        grid=(N // TM,),
        in_specs=[
            pl.BlockSpec((TM, HP), lambda i: (i, 0)),
            pl.BlockSpec((N, HP), lambda i: (0, 0)),
        ],
        out_specs=pl.BlockSpec((TM, N), lambda i: (i, 0)),
        compiler_params=pltpu.CompilerParams(
            dimension_semantics=("parallel",)),
        cost_estimate=pl.CostEstimate(
            flops=2 * N * N * HP, transcendentals=0,
            bytes_accessed=4 * N * N + 2 * 2 * N * HP),
    )(t_mat, t_mat)
    return out
